# Initial kernel scaffold; baseline (speedup 1.0000x reference)
import jax
import jax.numpy as jnp
from jax.experimental import pallas as pl


def kernel(x, edge_index, edge_attr, W0, b0, W1, b1):
    raise NotImplementedError("write your pallas kernel here")


# speedup vs baseline: 18.4018x; 18.4018x over previous
import functools

import jax
import jax.numpy as jnp
import numpy as np
from jax import lax
from jax.experimental import pallas as pl
from jax.experimental.pallas import tpu as pltpu
from jax.experimental.pallas import tpu_sc as plsc

N = 10000
E = 320000
D_IN = 128
D = 64
N_CLASSES = 32
T_DIFF = 5.0
K_TAYLOR = 10
EXP_NEG_T = float(np.exp(-T_DIFF))

NT = 16
ECH = 128
NP = 10240
NPT = NP // NT
RCH = 128
NCHK = NPT // RCH
EP = 321536
EPT = EP // NT
NCH = EPT // ECH
GB = 3
IB = 4
UNROLL = 12

_ZERO16 = functools.partial(jnp.zeros, (16,), jnp.int32)


def _sc_gcn_core(y0p, rowp, colp, attrp, b0, ratios):
    mesh = plsc.VectorSubcoreMesh(
        core_axis_name="c", subcore_axis_name="s", num_cores=1
    )

    @functools.partial(
        pl.kernel,
        out_type=jax.ShapeDtypeStruct((NP, D), jnp.float32),
        mesh=mesh,
        compiler_params=pltpu.CompilerParams(
            needs_layout_passes=False, use_tc_tiling_on_sc=False
        ),
        scratch_types=[
            pltpu.VMEM((IB, ECH), jnp.int32),
            pltpu.VMEM((IB, ECH), jnp.int32),
            pltpu.VMEM((IB, ECH), jnp.float32),
            pltpu.VMEM((GB, ECH, D), jnp.float32),
            pltpu.VMEM((2, RCH, D), jnp.float32),
            pltpu.VMEM((NPT, D), jnp.float32),
            pltpu.VMEM((NPT,), jnp.float32),
            pltpu.VMEM((NPT,), jnp.float32),
            pltpu.VMEM((NPT,), jnp.float32),
            pltpu.VMEM((NPT,), jnp.float32),
            pltpu.VMEM((ECH,), jnp.float32),
            pltpu.VMEM((D,), jnp.float32),
            pltpu.VMEM((16,), jnp.float32),
            pltpu.VMEM((16,), jnp.float32),
            pltpu.VMEM_SHARED((NP, D), jnp.float32),
            pltpu.VMEM_SHARED((NP,), jnp.float32),
            pltpu.VMEM_SHARED((NP,), jnp.float32),
            pltpu.SemaphoreType.DMA((IB,)),
            pltpu.SemaphoreType.DMA((IB,)),
            pltpu.SemaphoreType.DMA((IB,)),
            pltpu.SemaphoreType.DMA((GB,)),
            pltpu.SemaphoreType.DMA((GB,)),
            pltpu.SemaphoreType.DMA((GB,)),
            pltpu.SemaphoreType.DMA((2,)),
            pltpu.SemaphoreType.DMA((2,)),
        ],
    )
    def k(y0_hbm, row_hbm, col_hbm, attr_hbm, b0_hbm, ratio_hbm, out_hbm,
          rbuf, cbuf, abuf, gbuf, nbuf, accU_t,
          dinvu_t, dinvw_t, eu_t, rwu_t, ones_t, b0_t, coef_t, ratio_t,
          S_sh, DEGU, DEGW,
          rsem, csem, asem, gsem, ssem, s2sem, wsemU, wsemS):
        t = lax.axis_index("s")
        nbase = t * NPT

        pltpu.sync_copy(b0_hbm, b0_t)
        pltpu.sync_copy(ratio_hbm, ratio_t)
        coef_t[pl.ds(0, 16)] = jnp.full((16,), EXP_NEG_T, jnp.float32)

        @pl.loop(0, ECH, step=16)
        def _(i):
            ones_t[pl.ds(i, 16)] = jnp.full((16,), 1.0, jnp.float32)

        @pl.loop(0, NPT, step=16)
        def _(i):
            dinvu_t[pl.ds(i, 16)] = jnp.full((16,), 1.0, jnp.float32)
        pltpu.sync_copy(dinvu_t, DEGU.at[pl.ds(nbase, NPT)])
        pltpu.sync_copy(dinvu_t, DEGW.at[pl.ds(nbase, NPT)])
        plsc.subcore_barrier()

        for c in range(min(IB - 1, NCH)):
            pltpu.async_copy(col_hbm.at[t, c], cbuf.at[c % IB], csem.at[c % IB])
            pltpu.async_copy(attr_hbm.at[t, c], abuf.at[c % IB], asem.at[c % IB])

        @pl.loop(0, NCH, step=UNROLL)
        def _(i0):
            for kk in range(UNROLL):
                i = i0 + kk
                bi = kk % IB
                bg = kk % GB

                @pl.when(i < NCH)
                def _():
                    pltpu.make_async_copy(
                        col_hbm.at[t, 0], cbuf.at[bi], csem.at[bi]
                    ).wait()
                    pltpu.make_async_copy(
                        attr_hbm.at[t, 0], abuf.at[bi], asem.at[bi]
                    ).wait()
                    pltpu.async_copy(
                        ones_t,
                        DEGU.at[plsc.Indices(cbuf.at[bi])],
                        ssem.at[bg],
                        add=True,
                    )
                    pltpu.async_copy(
                        abuf.at[bi],
                        DEGW.at[plsc.Indices(cbuf.at[bi])],
                        s2sem.at[bg],
                        add=True,
                    )

                    @pl.when(i >= 1)
                    def _():
                        bp = (kk + GB - 1) % GB
                        pltpu.make_async_copy(
                            ones_t,
                            DEGU.at[plsc.Indices(cbuf.at[0])],
                            ssem.at[bp],
                        ).wait()
                        pltpu.make_async_copy(
                            abuf.at[0],
                            DEGW.at[plsc.Indices(cbuf.at[0])],
                            s2sem.at[bp],
                        ).wait()

                    @pl.when(i + IB - 1 < NCH)
                    def _():
                        bn = (kk + IB - 1) % IB
                        pltpu.async_copy(
                            col_hbm.at[t, i + IB - 1], cbuf.at[bn],
                            csem.at[bn],
                        )
                        pltpu.async_copy(
                            attr_hbm.at[t, i + IB - 1], abuf.at[bn],
                            asem.at[bn],
                        )

        b = (NCH - 1) % GB
        pltpu.make_async_copy(
            ones_t, DEGU.at[plsc.Indices(cbuf.at[0])], ssem.at[b]
        ).wait()
        pltpu.make_async_copy(
            abuf.at[0], DEGW.at[plsc.Indices(cbuf.at[0])], s2sem.at[b]
        ).wait()
        plsc.subcore_barrier()

        pltpu.sync_copy(DEGU.at[pl.ds(nbase, NPT)], dinvu_t)
        pltpu.sync_copy(DEGW.at[pl.ds(nbase, NPT)], dinvw_t)

        def _rsqrt16(d):
            i = lax.bitcast_convert_type(d, jnp.int32)
            i = jnp.int32(0x5F3759DF) - lax.shift_right_logical(i, 1)
            y = lax.bitcast_convert_type(i, jnp.float32)
            for _ in range(3):
                y = y * (1.5 - 0.5 * d * y * y)
            return y

        @pl.loop(0, NPT, step=16)
        def _(i):
            sl = pl.ds(i, 16)
            du = dinvu_t[sl]
            dw = dinvw_t[sl]
            yu = _rsqrt16(du)
            yw = _rsqrt16(dw)
            dinvu_t[sl] = yu
            dinvw_t[sl] = yw
            eu_t[sl] = yu * yu
            rwu_t[sl] = yw * du * yu

        def _bcast(ref, n):
            return plsc.load_gather(ref, [_ZERO16() + n])

        def node_phase(kind):
            cvec = coef_t[pl.ds(0, 16)]
            if kind == "init":
                scale_ref = dinvu_t
            elif kind == "td":
                scale_ref = eu_t
            elif kind == "l0pre":
                scale_ref = rwu_t
            else:
                scale_ref = dinvw_t
            for j in range(NCHK):
                b = j % 2
                base = nbase + j * RCH
                if j >= 2:
                    pltpu.make_async_copy(
                        nbuf.at[b], out_hbm.at[pl.ds(nbase, RCH)], wsemU.at[b]
                    ).wait()
                    if kind != "final":
                        pltpu.make_async_copy(
                            nbuf.at[b], S_sh.at[pl.ds(nbase, RCH)], wsemS.at[b]
                        ).wait()
                if kind == "init":
                    pltpu.sync_copy(y0_hbm.at[pl.ds(base, RCH)], nbuf.at[b])
                elif kind != "l0pre":
                    pltpu.sync_copy(S_sh.at[pl.ds(base, RCH)], nbuf.at[b])

                @pl.loop(0, RCH)
                def _(r):
                    n = j * RCH + r
                    sc = _bcast(scale_ref, n)
                    for q in range(D // 16):
                        sl = pl.ds(q * 16, 16)
                        if kind == "l0pre":
                            v = accU_t[n, sl] * sc
                        else:
                            v = nbuf[b, r, sl] * sc
                        if kind == "init":
                            accU_t[n, sl] = EXP_NEG_T * v
                        elif kind == "td":
                            accU_t[n, sl] = accU_t[n, sl] + cvec * v
                        elif kind == "l0":
                            v = sc * jnp.maximum(v + b0_t[sl], 0.0)
                        nbuf[b, r, sl] = v

                pltpu.async_copy(
                    nbuf.at[b], out_hbm.at[pl.ds(base, RCH)], wsemU.at[b]
                )
                if kind != "final":
                    pltpu.async_copy(
                        nbuf.at[b], S_sh.at[pl.ds(base, RCH)], wsemS.at[b]
                    )
            for j in (NCHK - 2, NCHK - 1):
                b = j % 2
                pltpu.make_async_copy(
                    nbuf.at[b], out_hbm.at[pl.ds(nbase, RCH)], wsemU.at[b]
                ).wait()
                if kind != "final":
                    pltpu.make_async_copy(
                        nbuf.at[b], S_sh.at[pl.ds(nbase, RCH)], wsemS.at[b]
                    ).wait()

        def edge_phase(weighted):
            for c in range(min(IB - 1, NCH)):
                bi = c % IB
                pltpu.async_copy(row_hbm.at[t, c], rbuf.at[bi], rsem.at[bi])
                pltpu.async_copy(col_hbm.at[t, c], cbuf.at[bi], csem.at[bi])
                if weighted:
                    pltpu.async_copy(
                        attr_hbm.at[t, c], abuf.at[bi], asem.at[bi]
                    )
            for c in range(min(GB - 1, NCH)):
                bg = c % GB
                bi = c % IB
                pltpu.make_async_copy(
                    row_hbm.at[t, 0], rbuf.at[bi], rsem.at[bi]
                ).wait()
                pltpu.async_copy(
                    out_hbm.at[plsc.Indices(rbuf.at[bi])],
                    gbuf.at[bg],
                    gsem.at[bg],
                )

            @pl.loop(0, NCH, step=UNROLL)
            def _(i0):
                for kk in range(UNROLL):
                    i = i0 + kk
                    bi = kk % IB
                    bg = kk % GB

                    @pl.when(i < NCH)
                    def _():
                        pltpu.make_async_copy(
                            out_hbm.at[plsc.Indices(rbuf.at[0])],
                            gbuf.at[bg],
                            gsem.at[bg],
                        ).wait()
                        pltpu.make_async_copy(
                            col_hbm.at[t, 0], cbuf.at[bi], csem.at[bi]
                        ).wait()
                        if weighted:
                            pltpu.make_async_copy(
                                attr_hbm.at[t, 0], abuf.at[bi], asem.at[bi]
                            ).wait()

                            @pl.loop(0, ECH)
                            def _(r):
                                av = plsc.load_gather(
                                    abuf, [_ZERO16() + bi, _ZERO16() + r]
                                )
                                for q in range(D // 16):
                                    sl = pl.ds(q * 16, 16)
                                    gbuf[bg, r, sl] = gbuf[bg, r, sl] * av

                        pltpu.async_copy(
                            gbuf.at[bg],
                            S_sh.at[plsc.Indices(cbuf.at[bi])],
                            ssem.at[bg],
                            add=True,
                        )

                        @pl.when(i >= 1)
                        def _():
                            bp = (kk + GB - 1) % GB
                            pltpu.make_async_copy(
                                gbuf.at[bp],
                                S_sh.at[plsc.Indices(cbuf.at[0])],
                                ssem.at[bp],
                            ).wait()

                        @pl.when(i + IB - 1 < NCH)
                        def _():
                            bn = (kk + IB - 1) % IB
                            pltpu.async_copy(
                                row_hbm.at[t, i + IB - 1], rbuf.at[bn],
                                rsem.at[bn],
                            )
                            pltpu.async_copy(
                                col_hbm.at[t, i + IB - 1], cbuf.at[bn],
                                csem.at[bn],
                            )
                            if weighted:
                                pltpu.async_copy(
                                    attr_hbm.at[t, i + IB - 1], abuf.at[bn],
                                    asem.at[bn],
                                )

                        @pl.when(i + GB - 1 < NCH)
                        def _():
                            bgn = (kk + GB - 1) % GB
                            bin_ = (kk + GB - 1) % IB
                            pltpu.make_async_copy(
                                row_hbm.at[t, 0], rbuf.at[bin_],
                                rsem.at[bin_],
                            ).wait()
                            pltpu.async_copy(
                                out_hbm.at[plsc.Indices(rbuf.at[bin_])],
                                gbuf.at[bgn],
                                gsem.at[bgn],
                            )

            b = (NCH - 1) % GB
            pltpu.make_async_copy(
                gbuf.at[b], S_sh.at[plsc.Indices(cbuf.at[0])], ssem.at[b]
            ).wait()

        node_phase("init")
        plsc.subcore_barrier()

        @pl.loop(1, K_TAYLOR + 1)
        def _(kk):
            edge_phase(False)
            plsc.subcore_barrier()
            rv = plsc.load_gather(ratio_t, [_ZERO16() + kk])
            coef_t[pl.ds(0, 16)] = coef_t[pl.ds(0, 16)] * rv
            node_phase("td")
            plsc.subcore_barrier()

        node_phase("l0pre")
        plsc.subcore_barrier()
        edge_phase(True)
        plsc.subcore_barrier()
        node_phase("l0")
        plsc.subcore_barrier()
        edge_phase(True)
        plsc.subcore_barrier()
        node_phase("final")

    return k(y0p, rowp, colp, attrp, b0, ratios)


def _tc_in_proj(xp, W0):
    def body(x_ref, w_ref, o_ref):
        o_ref[...] = jnp.dot(
            x_ref[...], w_ref[...], preferred_element_type=jnp.float32
        )

    return pl.pallas_call(
        body, out_shape=jax.ShapeDtypeStruct((NP, D), jnp.float32)
    )(xp, W0)


def _tc_head(p, W1, b1):
    def body(p_ref, w_ref, b_ref, o_ref):
        h = jnp.dot(
            p_ref[...], w_ref[...], preferred_element_type=jnp.float32
        ) + b_ref[...]
        m = jnp.max(h, axis=1, keepdims=True)
        s = h - m
        lse = jnp.log(jnp.sum(jnp.exp(s), axis=1, keepdims=True))
        o_ref[...] = s - lse

    return pl.pallas_call(
        body, out_shape=jax.ShapeDtypeStruct((N, N_CLASSES), jnp.float32)
    )(p, W1, b1.reshape(1, N_CLASSES))


def kernel(x, edge_index, edge_attr, W0, b0, W1, b1):
    xp = jnp.pad(x, ((0, NP - N), (0, 0)))
    y0p = _tc_in_proj(xp, W0)

    pad = EP - E
    rowp = jnp.concatenate(
        [edge_index[0], jnp.full((pad,), NP - 1, jnp.int32)]
    ).reshape(NT, NCH, ECH)
    colp = jnp.concatenate(
        [edge_index[1], jnp.full((pad,), NP - 1, jnp.int32)]
    ).reshape(NT, NCH, ECH)
    attrp = jnp.concatenate(
        [edge_attr, jnp.zeros((pad,), jnp.float32)]
    ).reshape(NT, NCH, ECH)

    ratios = jnp.asarray(
        np.array([1.0] + [T_DIFF / k for k in range(1, 16)], np.float32)
    )
    u = _sc_gcn_core(y0p, rowp, colp, attrp, b0, ratios)
    return _tc_head(u[:N], W1, b1)
